# baseline (speedup 1.0000x reference)
import functools

import jax
import jax.numpy as jnp
from jax import lax
from jax.experimental import pallas as pl
from jax.experimental.pallas import tpu as pltpu
from jax.experimental.pallas import tpu_sc as plsc

_MARGIN = 0.02
_EPS = 1e-5
_N = 100000
_D = 128
_E = _N - 1

_ROWS = 1024
_ROWS_LOG2 = 10
_N_FULL_CH = _N // _ROWS
_TAIL_ROWS = _N - _N_FULL_CH * _ROWS
_N_CH = _N_FULL_CH + 1
_TBL_COLS = 112
_GCOLS = 8
_G_ROWS = _GCOLS * _ROWS
_N_G = (_N + _G_ROWS - 1) // _G_ROWS
_NBUF = 3

_NC = 2
_NS = 16
_NW = _NC * _NS
_LANES = 16
_CHUNK = 3200
_E_PAD = _NW * _CHUNK
_UNROLL = 8


def _radii_body(x_hbm, o_ref, buf, sems):
    def copy(g):
        slot = g % _NBUF
        r0 = g * _G_ROWS
        rows = min(_G_ROWS, _N - r0)
        return pltpu.make_async_copy(
            x_hbm.at[pl.ds(r0, rows), :],
            buf.at[slot, pl.ds(0, rows)], sems.at[slot])

    for g in range(_NBUF - 1):
        copy(g).start()
    for g in range(_N_G):
        copy(g).wait()
        if g + _NBUF - 1 < _N_G:
            copy(g + _NBUF - 1).start()
        for k in range(_GCOLS):
            col = g * _GCOLS + k
            if col >= _N_CH:
                break
            rows = _ROWS if col < _N_FULL_CH else _TAIL_ROWS
            x = buf[g % _NBUF, pl.ds(k * _ROWS, rows)]
            ss = jnp.sum(x * x, axis=1, keepdims=True)
            r = jnp.minimum(jnp.sqrt(ss), 1.0 - _EPS)
            o_ref[pl.ds(0, rows), col:col + 1] = r


def _compute_radii(embeddings):
    return pl.pallas_call(
        _radii_body,
        in_specs=[pl.BlockSpec(memory_space=pl.ANY)],
        out_specs=pl.BlockSpec(memory_space=pltpu.VMEM),
        out_shape=jax.ShapeDtypeStruct((_ROWS, _D), jnp.float32),
        scratch_shapes=[
            pltpu.VMEM((_NBUF, _G_ROWS, _D), jnp.float32),
            pltpu.SemaphoreType.DMA((_NBUF,)),
        ],
    )(embeddings)


def _loss_body(radii_hbm, pidx_hbm, cidx_hbm, out_hbm, table_v, idx_v, acc_v):
    c = lax.axis_index("c")
    s = lax.axis_index("s")
    wid = s * _NC + c
    base = wid * _CHUNK

    pltpu.sync_copy(radii_hbm.at[:, pl.ds(0, _TBL_COLS)], table_v)
    pltpu.sync_copy(pidx_hbm.at[pl.ds(base, _CHUNK)], idx_v.at[pl.ds(0, _CHUNK)])
    pltpu.sync_copy(cidx_hbm.at[pl.ds(base, _CHUNK)],
                    idx_v.at[pl.ds(_CHUNK, _CHUNK)])

    iota = lax.iota(jnp.int32, _LANES)
    edge0 = wid * _CHUNK

    def step(j, acc):
        for u in range(_UNROLL):
            off = (j * _UNROLL + u) * _LANES
            pidx = idx_v[pl.ds(off, _LANES)]
            cidx = idx_v[pl.ds(_CHUNK + off, _LANES)]
            pv = plsc.load_gather(
                table_v,
                [jnp.bitwise_and(pidx, _ROWS - 1),
                 lax.shift_right_logical(pidx, _ROWS_LOG2)])
            cv = plsc.load_gather(
                table_v,
                [jnp.bitwise_and(cidx, _ROWS - 1),
                 lax.shift_right_logical(cidx, _ROWS_LOG2)])
            val = jnp.maximum(pv + _MARGIN - cv, 0.0)
            edge = edge0 + off + iota
            val = jnp.where(edge < _E, val, 0.0)
            acc = acc + val
        return acc

    acc = lax.fori_loop(0, _CHUNK // _LANES // _UNROLL, step,
                        jnp.zeros((_LANES,), jnp.float32))
    acc_v[...] = acc
    pltpu.sync_copy(acc_v, out_hbm.at[wid])


@functools.cache
def _make_loss_call():
    return pl.kernel(
        _loss_body,
        out_type=jax.ShapeDtypeStruct((_NW, _LANES), jnp.float32),
        mesh=plsc.VectorSubcoreMesh(core_axis_name="c", subcore_axis_name="s"),
        compiler_params=pltpu.CompilerParams(
            needs_layout_passes=False, use_tc_tiling_on_sc=False),
        scratch_types=[
            pltpu.VMEM((_ROWS, _TBL_COLS), jnp.float32),
            pltpu.VMEM((2 * _CHUNK,), jnp.int32),
            pltpu.VMEM((_LANES,), jnp.float32),
        ],
    )


def kernel(embeddings, child_indices, parent_indices):
    radii2d = _compute_radii(embeddings)
    pad = _E_PAD - _E
    pidx = jnp.pad(parent_indices, (0, pad))
    cidx = jnp.pad(child_indices, (0, pad))
    partials = _make_loss_call()(radii2d, pidx, cidx)
    return jnp.sum(partials) / _E


# speedup vs baseline: 1.0140x; 1.0140x over previous
import functools

import jax
import jax.numpy as jnp
from jax import lax
from jax.experimental import pallas as pl
from jax.experimental.pallas import tpu as pltpu
from jax.experimental.pallas import tpu_sc as plsc

_MARGIN = 0.02
_EPS = 1e-5
_N = 100000
_D = 128
_E = _N - 1

_ROWS = 1024
_ROWS_LOG2 = 10
_N_FULL_CH = _N // _ROWS
_TAIL_ROWS = _N - _N_FULL_CH * _ROWS
_N_CH = _N_FULL_CH + 1
_TBL_COLS = 112
_GCOLS = 8
_G_ROWS = _GCOLS * _ROWS
_N_G = (_N + _G_ROWS - 1) // _G_ROWS
_NBUF = 3

_NC = 2
_NS = 16
_NW = _NC * _NS
_LANES = 16
_CHUNK = 3200
_E_PAD = _NW * _CHUNK
_UNROLL = 8


def _radii_body(x_hbm, o_ref, buf, sems):
    def copy(g):
        slot = g % _NBUF
        r0 = g * _G_ROWS
        rows = min(_G_ROWS, _N - r0)
        return pltpu.make_async_copy(
            x_hbm.at[pl.ds(r0, rows), :],
            buf.at[slot, pl.ds(0, rows)], sems.at[slot])

    for g in range(_NBUF - 1):
        copy(g).start()
    for g in range(_N_G):
        copy(g).wait()
        if g + _NBUF - 1 < _N_G:
            copy(g + _NBUF - 1).start()
        for k in range(_GCOLS):
            col = g * _GCOLS + k
            if col >= _N_CH:
                break
            rows = _ROWS if col < _N_FULL_CH else _TAIL_ROWS
            x = buf[g % _NBUF, pl.ds(k * _ROWS, rows)]
            ss = jnp.sum(x * x, axis=1, keepdims=True)
            r = jnp.minimum(jnp.sqrt(ss), 1.0 - _EPS)
            o_ref[pl.ds(0, rows), col:col + 1] = r


def _compute_radii(embeddings):
    return pl.pallas_call(
        _radii_body,
        in_specs=[pl.BlockSpec(memory_space=pl.ANY)],
        out_specs=pl.BlockSpec(memory_space=pltpu.VMEM),
        out_shape=jax.ShapeDtypeStruct((_ROWS, _D), jnp.float32),
        scratch_shapes=[
            pltpu.VMEM((_NBUF, _G_ROWS, _D), jnp.float32),
            pltpu.SemaphoreType.DMA((_NBUF,)),
        ],
    )(embeddings)


def _loss_body(radii_hbm, idx_hbm, out_hbm, table_v, idx_v, acc_v):
    c = lax.axis_index("c")
    s = lax.axis_index("s")
    wid = s * _NC + c
    base = wid * (2 * _CHUNK)

    pltpu.sync_copy(radii_hbm.at[:, pl.ds(0, _TBL_COLS)], table_v)
    pltpu.sync_copy(idx_hbm.at[pl.ds(base, 2 * _CHUNK)], idx_v)

    iota = lax.iota(jnp.int32, _LANES)
    edge0 = wid * _CHUNK

    def step(j, acc):
        for u in range(_UNROLL):
            off = (j * _UNROLL + u) * _LANES
            pidx = idx_v[pl.ds(off, _LANES)]
            cidx = idx_v[pl.ds(_CHUNK + off, _LANES)]
            pv = plsc.load_gather(
                table_v,
                [jnp.bitwise_and(pidx, _ROWS - 1),
                 lax.shift_right_logical(pidx, _ROWS_LOG2)])
            cv = plsc.load_gather(
                table_v,
                [jnp.bitwise_and(cidx, _ROWS - 1),
                 lax.shift_right_logical(cidx, _ROWS_LOG2)])
            val = jnp.maximum(pv + _MARGIN - cv, 0.0)
            edge = edge0 + off + iota
            val = jnp.where(edge < _E, val, 0.0)
            acc = acc + val
        return acc

    acc = lax.fori_loop(0, _CHUNK // _LANES // _UNROLL, step,
                        jnp.zeros((_LANES,), jnp.float32))
    acc_v[...] = acc
    pltpu.sync_copy(acc_v, out_hbm.at[wid])


@functools.cache
def _make_loss_call():
    return pl.kernel(
        _loss_body,
        out_type=jax.ShapeDtypeStruct((_NW, _LANES), jnp.float32),
        mesh=plsc.VectorSubcoreMesh(core_axis_name="c", subcore_axis_name="s"),
        compiler_params=pltpu.CompilerParams(
            needs_layout_passes=False, use_tc_tiling_on_sc=False),
        scratch_types=[
            pltpu.VMEM((_ROWS, _TBL_COLS), jnp.float32),
            pltpu.VMEM((2 * _CHUNK,), jnp.int32),
            pltpu.VMEM((_LANES,), jnp.float32),
        ],
    )


def kernel(embeddings, child_indices, parent_indices):
    radii2d = _compute_radii(embeddings)
    pad = _E_PAD - _E
    pidx = jnp.pad(parent_indices, (0, pad)).reshape(_NW, _CHUNK)
    cidx = jnp.pad(child_indices, (0, pad)).reshape(_NW, _CHUNK)
    idxs = jnp.concatenate([pidx, cidx], axis=1).reshape(-1)
    partials = _make_loss_call()(radii2d, idxs)
    return jnp.sum(partials) / _E
